# baseline (speedup 1.0000x reference)
import functools

import jax
import jax.numpy as jnp
from jax import lax
from jax.experimental import pallas as pl
from jax.experimental.pallas import tpu as pltpu
from jax.experimental.pallas import tpu_sc as plsc

B = 16384
DIMS = [8, 8, 8, 2, 4, 3, 4, 4, 4]
VOCABS = [1000000, 100000, 100000, 3, 8, 4, 16, 8, 16]
SMALL_VOCABS = VOCABS[3:]
BIG_V = VOCABS[:3]
HIDDEN = 64

NC = 2
NS = 16
L = 16
NW = NC * NS
BPW = B // NW
CHUNK = 128
NCHUNK = BPW // CHUNK
NGRP = BPW // L
D8 = 8

NT = [-(-v // CHUNK) for v in BIG_V]
NTFULL = [v // CHUNK for v in BIG_V]
TAIL = [v % CHUNK for v in BIG_V]


def _sc_repack(t0t, t1t, t2t):
    mesh = plsc.VectorSubcoreMesh(core_axis_name="c", subcore_axis_name="s")

    @functools.partial(
        pl.kernel,
        mesh=mesh,
        compiler_params=pltpu.CompilerParams(use_tc_tiling_on_sc=True),
        out_type=[jax.ShapeDtypeStruct((nt, D8, CHUNK), jnp.float32)
                  for nt in NT],
        scratch_types=[
            pltpu.VMEM((D8, TAIL[0]), jnp.float32),
            pltpu.VMEM((D8, TAIL[1]), jnp.float32),
            pltpu.VMEM((D8, TAIL[2]), jnp.float32),
            pltpu.SemaphoreType.DMA,
        ],
    )
    def k(t0_hbm, t1_hbm, t2_hbm, f0, f1, f2, tb0, tb1, tb2, osem):
        wid = lax.axis_index("s") * NC + lax.axis_index("c")
        tabs = (t0_hbm, t1_hbm, t2_hbm)
        fouts = (f0, f1, f2)

        LAG = 8
        for f in range(3):
            nfull = NTFULL[f]
            nbase = nfull // NW
            nrem = nfull % NW
            nloc = jnp.where(wid < nrem, nbase + 1, nbase)
            start = wid * nbase + jnp.minimum(wid, nrem)

            def body(bl, f=f, nloc=nloc, start=start):
                c = start + bl
                off = pl.multiple_of(c * CHUNK, CHUNK)
                pltpu.async_copy(tabs[f].at[:, pl.ds(off, CHUNK)],
                                 fouts[f].at[c], osem)

                @pl.when(bl >= LAG)
                def _():
                    pltpu.make_async_copy(tabs[f].at[:, pl.ds(0, CHUNK)],
                                          fouts[f].at[0], osem).wait()
            pl.loop(0, nloc)(body)
            for kk in range(LAG):
                @pl.when(nloc > kk)
                def _(f=f):
                    pltpu.make_async_copy(tabs[f].at[:, pl.ds(0, CHUNK)],
                                          fouts[f].at[0], osem).wait()

        @pl.when(wid == 0)
        def _():
            tbufs = (tb0, tb1, tb2)
            for f in range(3):
                nfull, tail = NTFULL[f], TAIL[f]
                pltpu.sync_copy(tabs[f].at[:, pl.ds(nfull * CHUNK, tail)],
                                tbufs[f])
                handles = [
                    pltpu.async_copy(
                        tbufs[f].at[d],
                        fouts[f].at[nfull, d, pl.ds(0, tail)],
                        osem)
                    for d in range(D8)]
                for h in handles:
                    h.wait()

    return k(t0t, t1t, t2t)


def _sc_gather(xt, t0f, t1f, t2f):
    mesh = plsc.VectorSubcoreMesh(core_axis_name="c", subcore_axis_name="s")

    @functools.partial(
        pl.kernel,
        mesh=mesh,
        compiler_params=pltpu.CompilerParams(use_tc_tiling_on_sc=False,
                                             needs_layout_passes=False),
        out_type=[jax.ShapeDtypeStruct((D8, B), jnp.float32) for _ in range(3)],
        scratch_types=[
            pltpu.VMEM((BPW,), jnp.float32),
            pltpu.VMEM((BPW,), jnp.float32),
            pltpu.VMEM((BPW,), jnp.float32),
            pltpu.VMEM((D8, NCHUNK, CHUNK), jnp.int32),
            pltpu.VMEM((D8, NCHUNK, CHUNK), jnp.int32),
            pltpu.VMEM((D8, NCHUNK, CHUNK), jnp.int32),
            pltpu.VMEM((D8, BPW), jnp.float32),
            pltpu.VMEM((D8, BPW), jnp.float32),
            pltpu.VMEM((D8, BPW), jnp.float32),
            pltpu.SemaphoreType.DMA,
            pltpu.SemaphoreType.DMA,
            pltpu.SemaphoreType.DMA,
        ],
    )
    def k(xt_hbm, t0_hbm, t1_hbm, t2_hbm, o0, o1, o2,
          xf0, xf1, xf2, i0, i1, i2, g0, g1, g2, s0, s1, s2):
        wid = lax.axis_index("s") * NC + lax.axis_index("c")
        base = wid * BPW
        tabs = (t0_hbm, t1_hbm, t2_hbm)
        xfs = (xf0, xf1, xf2)
        idxs = (i0, i1, i2)
        gbufs = (g0, g1, g2)
        sems = (s0, s1, s2)
        outs = (o0, o1, o2)

        for f in range(3):
            pltpu.sync_copy(xt_hbm.at[f, pl.ds(base, BPW)], xfs[f])

        for f in range(3):
            def conv_body(g, f=f):
                v = xfs[f][pl.ds(g * L, L)].astype(jnp.int32)
                b2 = lax.shift_left(lax.shift_right_logical(v, 7), 10) + (v & 127)
                for d in range(D8):
                    idxs[f].at[d, g // 8][pl.ds((g % 8) * L, L)] = b2 + d * CHUNK
            pl.loop(0, NGRP)(conv_body)

        handles = []
        for f in range(3):
            for d in range(D8):
                for j in range(NCHUNK):
                    handles.append(pltpu.async_copy(
                        tabs[f].at[idxs[f].at[d, j]],
                        gbufs[f].at[d, pl.ds(j * CHUNK, CHUNK)],
                        sems[f]))
        for h in handles:
            h.wait()
        for f in range(3):
            pltpu.sync_copy(gbufs[f], outs[f].at[:, pl.ds(base, BPW)])

    return k(xt, t0f, t1f, t2f)


def _tc_mlp_body(xt, g0, g1, g2,
                 s0, s1, s2, s3, s4, s5, w1t, b1c, w2, b2, out):
    small = (s0, s1, s2, s3, s4, s5)
    z = lax.dot_general(w1t[0:8, :], g0[...], (((0,), (0,)), ((), ())),
                        preferred_element_type=jnp.float32)
    z = z + lax.dot_general(w1t[8:16, :], g1[...], (((0,), (0,)), ((), ())),
                            preferred_element_type=jnp.float32)
    z = z + lax.dot_general(w1t[16:24, :], g2[...], (((0,), (0,)), ((), ())),
                            preferred_element_type=jnp.float32)
    off = 24
    for f in range(6):
        v = SMALL_VOCABS[f]
        d = DIMS[3 + f]
        proj = lax.dot_general(small[f][...], w1t[off:off + d, :],
                               (((0,), (0,)), ((), ())),
                               preferred_element_type=jnp.float32)
        ids = xt[3 + f:4 + f, :].astype(jnp.int32)
        onehot = (lax.broadcasted_iota(jnp.int32, (v, 1), 0) == ids
                  ).astype(jnp.float32)
        z = z + lax.dot_general(proj, onehot, (((0,), (0,)), ((), ())),
                                preferred_element_type=jnp.float32)
        off += d
    z = z + lax.dot_general(w1t[45:46, :], xt[9:10, :],
                            (((0,), (0,)), ((), ())),
                            preferred_element_type=jnp.float32)
    z = z + b1c[...]
    h1 = jnp.maximum(z, 0.0)
    o = jnp.dot(w2[...], h1, preferred_element_type=jnp.float32) + b2[...]
    out[...] = jax.nn.sigmoid(o)


def kernel(x, W_emb0, W_emb1, W_emb2, W_emb3, W_emb4, W_emb5, W_emb6,
           W_emb7, W_emb8, W1, b1, W2, b2):
    xt = x.T
    r0, r1, r2 = _sc_repack(W_emb0.T, W_emb1.T, W_emb2.T)
    g0, g1, g2 = _sc_gather(xt, r0.reshape(-1), r1.reshape(-1), r2.reshape(-1))

    w1t = W1.T
    BB = 2048
    col_blk = lambda h: pl.BlockSpec((h, BB), lambda i: (0, i))
    full = lambda s: pl.BlockSpec(s, lambda i: (0, 0))
    out = pl.pallas_call(
        _tc_mlp_body,
        grid=(B // BB,),
        in_specs=[col_blk(10), col_blk(8), col_blk(8), col_blk(8),
                  full((2, 3)), full((4, 8)), full((3, 4)), full((4, 16)),
                  full((4, 8)), full((4, 16)),
                  full((46, HIDDEN)), full((HIDDEN, 1)), full((1, HIDDEN)),
                  full((1, 1))],
        out_specs=col_blk(1),
        out_shape=jax.ShapeDtypeStruct((1, B), jnp.float32),
    )(xt, g0, g1, g2,
      W_emb3.T, W_emb4.T, W_emb5.T, W_emb6.T, W_emb7.T, W_emb8.T,
      w1t, b1.reshape(HIDDEN, 1), W2, b2.reshape(1, 1))
    return jnp.reshape(out, (B,))


# speedup vs baseline: 10.8600x; 10.8600x over previous
import functools

import jax
import jax.numpy as jnp
from jax import lax
from jax.experimental import pallas as pl
from jax.experimental.pallas import tpu as pltpu
from jax.experimental.pallas import tpu_sc as plsc

B = 16384
DIMS = [8, 8, 8, 2, 4, 3, 4, 4, 4]
VOCABS = [1000000, 100000, 100000, 3, 8, 4, 16, 8, 16]
SMALL_VOCABS = VOCABS[3:]
BIG_V = VOCABS[:3]
HIDDEN = 64

NC = 2
NS = 16
L = 16
NW = NC * NS
BPW = B // NW
CHUNK = 128
NCHUNK = BPW // CHUNK
NGRP = BPW // L
D8 = 8

NT = [-(-v // CHUNK) for v in BIG_V]
NTFULL = [v // CHUNK for v in BIG_V]
TAIL = [v % CHUNK for v in BIG_V]
UT = 16
UNIT = UT * CHUNK
NU = [n // UT for n in NTFULL]
REMT = [n - u * UT for n, u in zip(NTFULL, NU)]


def _sc_repack(t0t, t1t, t2t):
    mesh = plsc.VectorSubcoreMesh(core_axis_name="c", subcore_axis_name="s")

    @functools.partial(
        pl.kernel,
        mesh=mesh,
        compiler_params=pltpu.CompilerParams(use_tc_tiling_on_sc=True),
        out_type=[jax.ShapeDtypeStruct((nt, D8, CHUNK), jnp.float32)
                  for nt in NT],
        scratch_types=[
            pltpu.VMEM((2, D8, UNIT), jnp.float32),
            pltpu.VMEM((D8, REMT[0] * CHUNK), jnp.float32),
            pltpu.VMEM((D8, REMT[1] * CHUNK), jnp.float32),
            pltpu.VMEM((D8, REMT[2] * CHUNK), jnp.float32),
            pltpu.VMEM((D8, TAIL[0]), jnp.float32),
            pltpu.VMEM((D8, TAIL[1]), jnp.float32),
            pltpu.VMEM((D8, TAIL[2]), jnp.float32),
            pltpu.SemaphoreType.DMA,
            pltpu.SemaphoreType.DMA,
        ],
    )
    def k(t0_hbm, t1_hbm, t2_hbm, f0, f1, f2, buf,
          rb0, rb1, rb2, tb0, tb1, tb2, isem, osem):
        wid = lax.axis_index("s") * NC + lax.axis_index("c")
        tabs = (t0_hbm, t1_hbm, t2_hbm)
        fouts = (f0, f1, f2)

        def fire_in(f, u, p):
            off = pl.multiple_of(u * UNIT, CHUNK)
            pltpu.async_copy(tabs[f].at[:, pl.ds(off, UNIT)],
                             buf.at[p], isem)

        def wait_in(f, p):
            pltpu.make_async_copy(tabs[f].at[:, pl.ds(0, UNIT)],
                                  buf.at[p], isem).wait()

        def fire_out(f, u, p):
            for t in range(UT):
                pltpu.async_copy(buf.at[p, :, pl.ds(t * CHUNK, CHUNK)],
                                 fouts[f].at[u * UT + t], osem)

        def wait_out(f):
            for t in range(UT):
                pltpu.make_async_copy(buf.at[0, :, pl.ds(0, CHUNK)],
                                      fouts[f].at[0], osem).wait()

        for f in range(3):
            nunit = NU[f]
            nbase = nunit // NW
            nrem = nunit % NW
            nloc = jnp.where(wid < nrem, nbase + 1, nbase)
            start = wid * nbase + jnp.minimum(wid, nrem)

            @pl.when(nloc > 0)
            def _(f=f, nloc=nloc, start=start):
                fire_in(f, start, 0)

                def body(bl, f=f, nloc=nloc, start=start):
                    p = lax.rem(bl, 2)
                    u = start + bl
                    wait_in(f, p)

                    @pl.when(bl >= 1)
                    def _():
                        wait_out(f)

                    @pl.when(bl + 1 < nloc)
                    def _():
                        fire_in(f, u + 1, 1 - p)

                    fire_out(f, u, p)
                pl.loop(0, nloc)(body)
                wait_out(f)

        @pl.when(wid == 0)
        def _():
            rbufs = (rb0, rb1, rb2)
            tbufs = (tb0, tb1, tb2)
            for f in range(3):
                nfull, tail, remt = NTFULL[f], TAIL[f], REMT[f]
                pltpu.sync_copy(
                    tabs[f].at[:, pl.ds(NU[f] * UNIT, remt * CHUNK)],
                    rbufs[f])
                pltpu.sync_copy(tabs[f].at[:, pl.ds(nfull * CHUNK, tail)],
                                tbufs[f])
                handles = []
                for t in range(remt):
                    handles.append(pltpu.async_copy(
                        rbufs[f].at[:, pl.ds(t * CHUNK, CHUNK)],
                        fouts[f].at[NU[f] * UT + t], osem))
                for d in range(D8):
                    handles.append(pltpu.async_copy(
                        tbufs[f].at[d],
                        fouts[f].at[nfull, d, pl.ds(0, tail)],
                        osem))
                for h in handles:
                    h.wait()

    return k(t0t, t1t, t2t)


def _sc_gather(xt, t0f, t1f, t2f):
    mesh = plsc.VectorSubcoreMesh(core_axis_name="c", subcore_axis_name="s")

    @functools.partial(
        pl.kernel,
        mesh=mesh,
        compiler_params=pltpu.CompilerParams(use_tc_tiling_on_sc=False,
                                             needs_layout_passes=False),
        out_type=[jax.ShapeDtypeStruct((D8, B), jnp.float32) for _ in range(3)],
        scratch_types=[
            pltpu.VMEM((BPW,), jnp.float32),
            pltpu.VMEM((BPW,), jnp.float32),
            pltpu.VMEM((BPW,), jnp.float32),
            pltpu.VMEM((D8, NCHUNK, CHUNK), jnp.int32),
            pltpu.VMEM((D8, NCHUNK, CHUNK), jnp.int32),
            pltpu.VMEM((D8, NCHUNK, CHUNK), jnp.int32),
            pltpu.VMEM((D8, BPW), jnp.float32),
            pltpu.VMEM((D8, BPW), jnp.float32),
            pltpu.VMEM((D8, BPW), jnp.float32),
            pltpu.SemaphoreType.DMA,
            pltpu.SemaphoreType.DMA,
            pltpu.SemaphoreType.DMA,
        ],
    )
    def k(xt_hbm, t0_hbm, t1_hbm, t2_hbm, o0, o1, o2,
          xf0, xf1, xf2, i0, i1, i2, g0, g1, g2, s0, s1, s2):
        wid = lax.axis_index("s") * NC + lax.axis_index("c")
        base = wid * BPW
        tabs = (t0_hbm, t1_hbm, t2_hbm)
        xfs = (xf0, xf1, xf2)
        idxs = (i0, i1, i2)
        gbufs = (g0, g1, g2)
        sems = (s0, s1, s2)
        outs = (o0, o1, o2)

        for f in range(3):
            pltpu.sync_copy(xt_hbm.at[f, pl.ds(base, BPW)], xfs[f])

        for f in range(3):
            def conv_body(g, f=f):
                v = xfs[f][pl.ds(g * L, L)].astype(jnp.int32)
                b2 = lax.shift_left(lax.shift_right_logical(v, 7), 10) + (v & 127)
                for d in range(D8):
                    idxs[f].at[d, g // 8][pl.ds((g % 8) * L, L)] = b2 + d * CHUNK
            pl.loop(0, NGRP)(conv_body)

        handles = []
        for f in range(3):
            for d in range(D8):
                for j in range(NCHUNK):
                    handles.append(pltpu.async_copy(
                        tabs[f].at[idxs[f].at[d, j]],
                        gbufs[f].at[d, pl.ds(j * CHUNK, CHUNK)],
                        sems[f]))
        for h in handles:
            h.wait()
        for f in range(3):
            pltpu.sync_copy(gbufs[f], outs[f].at[:, pl.ds(base, BPW)])

    return k(xt, t0f, t1f, t2f)


def _tc_mlp_body(xt, g0, g1, g2,
                 s0, s1, s2, s3, s4, s5, w1t, b1c, w2, b2, out):
    small = (s0, s1, s2, s3, s4, s5)
    z = lax.dot_general(w1t[0:8, :], g0[...], (((0,), (0,)), ((), ())),
                        preferred_element_type=jnp.float32)
    z = z + lax.dot_general(w1t[8:16, :], g1[...], (((0,), (0,)), ((), ())),
                            preferred_element_type=jnp.float32)
    z = z + lax.dot_general(w1t[16:24, :], g2[...], (((0,), (0,)), ((), ())),
                            preferred_element_type=jnp.float32)
    off = 24
    for f in range(6):
        v = SMALL_VOCABS[f]
        d = DIMS[3 + f]
        proj = lax.dot_general(small[f][...], w1t[off:off + d, :],
                               (((0,), (0,)), ((), ())),
                               preferred_element_type=jnp.float32)
        ids = xt[3 + f:4 + f, :].astype(jnp.int32)
        onehot = (lax.broadcasted_iota(jnp.int32, (v, 1), 0) == ids
                  ).astype(jnp.float32)
        z = z + lax.dot_general(proj, onehot, (((0,), (0,)), ((), ())),
                                preferred_element_type=jnp.float32)
        off += d
    z = z + lax.dot_general(w1t[45:46, :], xt[9:10, :],
                            (((0,), (0,)), ((), ())),
                            preferred_element_type=jnp.float32)
    z = z + b1c[...]
    h1 = jnp.maximum(z, 0.0)
    o = jnp.dot(w2[...], h1, preferred_element_type=jnp.float32) + b2[...]
    out[...] = jax.nn.sigmoid(o)


def kernel(x, W_emb0, W_emb1, W_emb2, W_emb3, W_emb4, W_emb5, W_emb6,
           W_emb7, W_emb8, W1, b1, W2, b2):
    xt = x.T
    r0, r1, r2 = _sc_repack(W_emb0.T, W_emb1.T, W_emb2.T)
    g0, g1, g2 = _sc_gather(xt, r0.reshape(-1), r1.reshape(-1), r2.reshape(-1))

    w1t = W1.T
    BB = 2048
    col_blk = lambda h: pl.BlockSpec((h, BB), lambda i: (0, i))
    full = lambda s: pl.BlockSpec(s, lambda i: (0, 0))
    out = pl.pallas_call(
        _tc_mlp_body,
        grid=(B // BB,),
        in_specs=[col_blk(10), col_blk(8), col_blk(8), col_blk(8),
                  full((2, 3)), full((4, 8)), full((3, 4)), full((4, 16)),
                  full((4, 8)), full((4, 16)),
                  full((46, HIDDEN)), full((HIDDEN, 1)), full((1, HIDDEN)),
                  full((1, 1))],
        out_specs=col_blk(1),
        out_shape=jax.ShapeDtypeStruct((1, B), jnp.float32),
    )(xt, g0, g1, g2,
      W_emb3.T, W_emb4.T, W_emb5.T, W_emb6.T, W_emb7.T, W_emb8.T,
      w1t, b1.reshape(HIDDEN, 1), W2, b2.reshape(1, 1))
    return jnp.reshape(out, (B,))


# speedup vs baseline: 11.4285x; 1.0523x over previous
import functools

import jax
import jax.numpy as jnp
from jax import lax
from jax.experimental import pallas as pl
from jax.experimental.pallas import tpu as pltpu
from jax.experimental.pallas import tpu_sc as plsc

B = 16384
DIMS = [8, 8, 8, 2, 4, 3, 4, 4, 4]
VOCABS = [1000000, 100000, 100000, 3, 8, 4, 16, 8, 16]
SMALL_VOCABS = VOCABS[3:]
BIG_V = VOCABS[:3]
HIDDEN = 64

NC = 2
NS = 16
L = 16
NW = NC * NS
BPW = B // NW
CHUNK = 128
NCHUNK = BPW // CHUNK
NGRP = BPW // L
D8 = 8

NT = [-(-v // CHUNK) for v in BIG_V]
NTFULL = [v // CHUNK for v in BIG_V]
TAIL = [v % CHUNK for v in BIG_V]
UT = 32
UNIT = UT * CHUNK
NU = [n // UT for n in NTFULL]
REMT = [n - u * UT for n, u in zip(NTFULL, NU)]


def _sc_repack(t0t, t1t, t2t):
    mesh = plsc.VectorSubcoreMesh(core_axis_name="c", subcore_axis_name="s")

    @functools.partial(
        pl.kernel,
        mesh=mesh,
        compiler_params=pltpu.CompilerParams(use_tc_tiling_on_sc=True),
        out_type=[jax.ShapeDtypeStruct((nt, D8, CHUNK), jnp.float32)
                  for nt in NT],
        scratch_types=[
            pltpu.VMEM((2, D8, UNIT), jnp.float32),
            pltpu.VMEM((D8, REMT[0] * CHUNK), jnp.float32),
            pltpu.VMEM((D8, REMT[1] * CHUNK), jnp.float32),
            pltpu.VMEM((D8, REMT[2] * CHUNK), jnp.float32),
            pltpu.VMEM((D8, TAIL[0]), jnp.float32),
            pltpu.VMEM((D8, TAIL[1]), jnp.float32),
            pltpu.VMEM((D8, TAIL[2]), jnp.float32),
            pltpu.SemaphoreType.DMA,
            pltpu.SemaphoreType.DMA,
        ],
    )
    def k(t0_hbm, t1_hbm, t2_hbm, f0, f1, f2, buf,
          rb0, rb1, rb2, tb0, tb1, tb2, isem, osem):
        wid = lax.axis_index("s") * NC + lax.axis_index("c")
        tabs = (t0_hbm, t1_hbm, t2_hbm)
        fouts = (f0, f1, f2)

        def fire_in(f, u, p):
            off = pl.multiple_of(u * UNIT, CHUNK)
            pltpu.async_copy(tabs[f].at[:, pl.ds(off, UNIT)],
                             buf.at[p], isem)

        def wait_in(f, p):
            pltpu.make_async_copy(tabs[f].at[:, pl.ds(0, UNIT)],
                                  buf.at[p], isem).wait()

        def fire_out(f, u, p):
            for t in range(UT):
                pltpu.async_copy(buf.at[p, :, pl.ds(t * CHUNK, CHUNK)],
                                 fouts[f].at[u * UT + t], osem)

        def wait_out(f):
            for t in range(UT):
                pltpu.make_async_copy(buf.at[0, :, pl.ds(0, CHUNK)],
                                      fouts[f].at[0], osem).wait()

        for f in range(3):
            nunit = NU[f]
            nbase = nunit // NW
            nrem = nunit % NW
            nloc = jnp.where(wid < nrem, nbase + 1, nbase)
            start = wid * nbase + jnp.minimum(wid, nrem)

            @pl.when(nloc > 0)
            def _(f=f, nloc=nloc, start=start):
                fire_in(f, start, 0)

                def body(bl, f=f, nloc=nloc, start=start):
                    p = lax.rem(bl, 2)
                    u = start + bl
                    wait_in(f, p)

                    @pl.when(bl >= 1)
                    def _():
                        wait_out(f)

                    @pl.when(bl + 1 < nloc)
                    def _():
                        fire_in(f, u + 1, 1 - p)

                    fire_out(f, u, p)
                pl.loop(0, nloc)(body)
                wait_out(f)

        @pl.when(wid == 0)
        def _():
            rbufs = (rb0, rb1, rb2)
            tbufs = (tb0, tb1, tb2)
            for f in range(3):
                nfull, tail, remt = NTFULL[f], TAIL[f], REMT[f]
                pltpu.sync_copy(
                    tabs[f].at[:, pl.ds(NU[f] * UNIT, remt * CHUNK)],
                    rbufs[f])
                pltpu.sync_copy(tabs[f].at[:, pl.ds(nfull * CHUNK, tail)],
                                tbufs[f])
                handles = []
                for t in range(remt):
                    handles.append(pltpu.async_copy(
                        rbufs[f].at[:, pl.ds(t * CHUNK, CHUNK)],
                        fouts[f].at[NU[f] * UT + t], osem))
                for d in range(D8):
                    handles.append(pltpu.async_copy(
                        tbufs[f].at[d],
                        fouts[f].at[nfull, d, pl.ds(0, tail)],
                        osem))
                for h in handles:
                    h.wait()

    return k(t0t, t1t, t2t)


def _sc_gather(xt, t0f, t1f, t2f):
    mesh = plsc.VectorSubcoreMesh(core_axis_name="c", subcore_axis_name="s")

    @functools.partial(
        pl.kernel,
        mesh=mesh,
        compiler_params=pltpu.CompilerParams(use_tc_tiling_on_sc=False,
                                             needs_layout_passes=False),
        out_type=[jax.ShapeDtypeStruct((D8, B), jnp.float32) for _ in range(3)],
        scratch_types=[
            pltpu.VMEM((BPW,), jnp.float32),
            pltpu.VMEM((BPW,), jnp.float32),
            pltpu.VMEM((BPW,), jnp.float32),
            pltpu.VMEM((D8, NCHUNK, CHUNK), jnp.int32),
            pltpu.VMEM((D8, NCHUNK, CHUNK), jnp.int32),
            pltpu.VMEM((D8, NCHUNK, CHUNK), jnp.int32),
            pltpu.VMEM((D8, BPW), jnp.float32),
            pltpu.VMEM((D8, BPW), jnp.float32),
            pltpu.VMEM((D8, BPW), jnp.float32),
            pltpu.SemaphoreType.DMA,
            pltpu.SemaphoreType.DMA,
            pltpu.SemaphoreType.DMA,
        ],
    )
    def k(xt_hbm, t0_hbm, t1_hbm, t2_hbm, o0, o1, o2,
          xf0, xf1, xf2, i0, i1, i2, g0, g1, g2, s0, s1, s2):
        wid = lax.axis_index("s") * NC + lax.axis_index("c")
        base = wid * BPW
        tabs = (t0_hbm, t1_hbm, t2_hbm)
        xfs = (xf0, xf1, xf2)
        idxs = (i0, i1, i2)
        gbufs = (g0, g1, g2)
        sems = (s0, s1, s2)
        outs = (o0, o1, o2)

        for f in range(3):
            pltpu.sync_copy(xt_hbm.at[f, pl.ds(base, BPW)], xfs[f])

        for f in range(3):
            def conv_body(g, f=f):
                v = xfs[f][pl.ds(g * L, L)].astype(jnp.int32)
                b2 = lax.shift_left(lax.shift_right_logical(v, 7), 10) + (v & 127)
                for d in range(D8):
                    idxs[f].at[d, g // 8][pl.ds((g % 8) * L, L)] = b2 + d * CHUNK
            pl.loop(0, NGRP)(conv_body)

        handles = []
        for f in range(3):
            for d in range(D8):
                for j in range(NCHUNK):
                    handles.append(pltpu.async_copy(
                        tabs[f].at[idxs[f].at[d, j]],
                        gbufs[f].at[d, pl.ds(j * CHUNK, CHUNK)],
                        sems[f]))
        for h in handles:
            h.wait()
        for f in range(3):
            pltpu.sync_copy(gbufs[f], outs[f].at[:, pl.ds(base, BPW)])

    return k(xt, t0f, t1f, t2f)


def _tc_mlp_body(xt, g0, g1, g2,
                 s0, s1, s2, s3, s4, s5, w1t, b1c, w2, b2, out):
    small = (s0, s1, s2, s3, s4, s5)
    z = lax.dot_general(w1t[0:8, :], g0[...], (((0,), (0,)), ((), ())),
                        preferred_element_type=jnp.float32)
    z = z + lax.dot_general(w1t[8:16, :], g1[...], (((0,), (0,)), ((), ())),
                            preferred_element_type=jnp.float32)
    z = z + lax.dot_general(w1t[16:24, :], g2[...], (((0,), (0,)), ((), ())),
                            preferred_element_type=jnp.float32)
    off = 24
    for f in range(6):
        v = SMALL_VOCABS[f]
        d = DIMS[3 + f]
        proj = lax.dot_general(small[f][...], w1t[off:off + d, :],
                               (((0,), (0,)), ((), ())),
                               preferred_element_type=jnp.float32)
        ids = xt[3 + f:4 + f, :].astype(jnp.int32)
        onehot = (lax.broadcasted_iota(jnp.int32, (v, 1), 0) == ids
                  ).astype(jnp.float32)
        z = z + lax.dot_general(proj, onehot, (((0,), (0,)), ((), ())),
                                preferred_element_type=jnp.float32)
        off += d
    z = z + lax.dot_general(w1t[45:46, :], xt[9:10, :],
                            (((0,), (0,)), ((), ())),
                            preferred_element_type=jnp.float32)
    z = z + b1c[...]
    h1 = jnp.maximum(z, 0.0)
    o = jnp.dot(w2[...], h1, preferred_element_type=jnp.float32) + b2[...]
    out[...] = jax.nn.sigmoid(o)


def kernel(x, W_emb0, W_emb1, W_emb2, W_emb3, W_emb4, W_emb5, W_emb6,
           W_emb7, W_emb8, W1, b1, W2, b2):
    xt = x.T
    r0, r1, r2 = _sc_repack(W_emb0.T, W_emb1.T, W_emb2.T)
    g0, g1, g2 = _sc_gather(xt, r0.reshape(-1), r1.reshape(-1), r2.reshape(-1))

    w1t = W1.T
    BB = 2048
    col_blk = lambda h: pl.BlockSpec((h, BB), lambda i: (0, i))
    full = lambda s: pl.BlockSpec(s, lambda i: (0, 0))
    out = pl.pallas_call(
        _tc_mlp_body,
        grid=(B // BB,),
        in_specs=[col_blk(10), col_blk(8), col_blk(8), col_blk(8),
                  full((2, 3)), full((4, 8)), full((3, 4)), full((4, 16)),
                  full((4, 8)), full((4, 16)),
                  full((46, HIDDEN)), full((HIDDEN, 1)), full((1, HIDDEN)),
                  full((1, 1))],
        out_specs=col_blk(1),
        out_shape=jax.ShapeDtypeStruct((1, B), jnp.float32),
    )(xt, g0, g1, g2,
      W_emb3.T, W_emb4.T, W_emb5.T, W_emb6.T, W_emb7.T, W_emb8.T,
      w1t, b1.reshape(HIDDEN, 1), W2, b2.reshape(1, 1))
    return jnp.reshape(out, (B,))


# speedup vs baseline: 12.8319x; 1.1228x over previous
import functools

import jax
import jax.numpy as jnp
from jax import lax
from jax.experimental import pallas as pl
from jax.experimental.pallas import tpu as pltpu
from jax.experimental.pallas import tpu_sc as plsc

B = 16384
DIMS = [8, 8, 8, 2, 4, 3, 4, 4, 4]
VOCABS = [1000000, 100000, 100000, 3, 8, 4, 16, 8, 16]
SMALL_VOCABS = VOCABS[3:]
BIG_V = VOCABS[:3]
HIDDEN = 64

NC = 2
NS = 16
L = 16
NW = NC * NS
BPW = B // NW
CHUNK = 128
NCHUNK = BPW // CHUNK
NGRP = BPW // L
D8 = 8

NT = [-(-v // CHUNK) for v in BIG_V]
NTFULL = [v // CHUNK for v in BIG_V]
TAIL = [v % CHUNK for v in BIG_V]
UT = 32
UNIT = UT * CHUNK
NU = [n // UT for n in NTFULL]
REMT = [n - u * UT for n, u in zip(NTFULL, NU)]


def _sc_repack(t0t, t1t, t2t):
    mesh = plsc.VectorSubcoreMesh(core_axis_name="c", subcore_axis_name="s")

    @functools.partial(
        pl.kernel,
        mesh=mesh,
        compiler_params=pltpu.CompilerParams(use_tc_tiling_on_sc=True),
        out_type=[jax.ShapeDtypeStruct((nt, D8, CHUNK), jnp.float32)
                  for nt in NT],
        scratch_types=[
            pltpu.VMEM((2, D8, UNIT), jnp.float32),
            pltpu.VMEM((D8, REMT[0] * CHUNK), jnp.float32),
            pltpu.VMEM((D8, REMT[1] * CHUNK), jnp.float32),
            pltpu.VMEM((D8, REMT[2] * CHUNK), jnp.float32),
            pltpu.VMEM((D8, TAIL[0]), jnp.float32),
            pltpu.VMEM((D8, TAIL[1]), jnp.float32),
            pltpu.VMEM((D8, TAIL[2]), jnp.float32),
            pltpu.SemaphoreType.DMA,
            pltpu.SemaphoreType.DMA,
        ],
    )
    def k(t0_hbm, t1_hbm, t2_hbm, f0, f1, f2, buf,
          rb0, rb1, rb2, tb0, tb1, tb2, isem, osem):
        wid = lax.axis_index("s") * NC + lax.axis_index("c")
        tabs = (t0_hbm, t1_hbm, t2_hbm)
        fouts = (f0, f1, f2)

        def fire_in(f, u, p):
            off = pl.multiple_of(u * UNIT, CHUNK)
            pltpu.async_copy(tabs[f].at[:, pl.ds(off, UNIT)],
                             buf.at[p], isem)

        def wait_in(f, p):
            pltpu.make_async_copy(tabs[f].at[:, pl.ds(0, UNIT)],
                                  buf.at[p], isem).wait()

        def fire_out(f, u, p):
            for t in range(UT):
                pltpu.async_copy(buf.at[p, :, pl.ds(t * CHUNK, CHUNK)],
                                 fouts[f].at[u * UT + t], osem)

        def wait_out(f):
            for t in range(UT):
                pltpu.make_async_copy(buf.at[0, :, pl.ds(0, CHUNK)],
                                      fouts[f].at[0], osem).wait()

        for f in range(3):
            nunit = NU[f]
            nbase = nunit // NW
            nrem = nunit % NW
            nloc = jnp.where(wid < nrem, nbase + 1, nbase)
            start = wid * nbase + jnp.minimum(wid, nrem)

            @pl.when(nloc > 0)
            def _(f=f, nloc=nloc, start=start):
                fire_in(f, start, 0)

                def body(bl, f=f, nloc=nloc, start=start):
                    p = lax.rem(bl, 2)
                    u = start + bl
                    wait_in(f, p)

                    @pl.when(bl >= 1)
                    def _():
                        wait_out(f)

                    @pl.when(bl + 1 < nloc)
                    def _():
                        fire_in(f, u + 1, 1 - p)

                    fire_out(f, u, p)
                pl.loop(0, nloc)(body)
                wait_out(f)

        @pl.when(wid == 0)
        def _():
            rbufs = (rb0, rb1, rb2)
            tbufs = (tb0, tb1, tb2)
            for f in range(3):
                nfull, tail, remt = NTFULL[f], TAIL[f], REMT[f]
                pltpu.sync_copy(
                    tabs[f].at[:, pl.ds(NU[f] * UNIT, remt * CHUNK)],
                    rbufs[f])
                pltpu.sync_copy(tabs[f].at[:, pl.ds(nfull * CHUNK, tail)],
                                tbufs[f])
                handles = []
                for t in range(remt):
                    handles.append(pltpu.async_copy(
                        rbufs[f].at[:, pl.ds(t * CHUNK, CHUNK)],
                        fouts[f].at[NU[f] * UT + t], osem))
                for d in range(D8):
                    handles.append(pltpu.async_copy(
                        tbufs[f].at[d],
                        fouts[f].at[nfull, d, pl.ds(0, tail)],
                        osem))
                for h in handles:
                    h.wait()

    return k(t0t, t1t, t2t)


def _sc_gather(xt, t0f, t1f, t2f):
    mesh = plsc.VectorSubcoreMesh(core_axis_name="c", subcore_axis_name="s")

    @functools.partial(
        pl.kernel,
        mesh=mesh,
        compiler_params=pltpu.CompilerParams(use_tc_tiling_on_sc=False,
                                             needs_layout_passes=False),
        out_type=[jax.ShapeDtypeStruct((D8, B), jnp.float32) for _ in range(3)],
        scratch_types=[
            pltpu.VMEM((BPW,), jnp.float32),
            pltpu.VMEM((BPW,), jnp.float32),
            pltpu.VMEM((BPW,), jnp.float32),
            pltpu.VMEM((D8, NCHUNK, CHUNK), jnp.int32),
            pltpu.VMEM((D8, NCHUNK, CHUNK), jnp.int32),
            pltpu.VMEM((D8, NCHUNK, CHUNK), jnp.int32),
            pltpu.VMEM((D8, BPW), jnp.float32),
            pltpu.VMEM((D8, BPW), jnp.float32),
            pltpu.VMEM((D8, BPW), jnp.float32),
            pltpu.SemaphoreType.DMA,
            pltpu.SemaphoreType.DMA,
            pltpu.SemaphoreType.DMA,
        ],
    )
    def k(xt_hbm, t0_hbm, t1_hbm, t2_hbm, o0, o1, o2,
          xf0, xf1, xf2, i0, i1, i2, g0, g1, g2, s0, s1, s2):
        wid = lax.axis_index("s") * NC + lax.axis_index("c")
        base = wid * BPW
        tabs = (t0_hbm, t1_hbm, t2_hbm)
        xfs = (xf0, xf1, xf2)
        idxs = (i0, i1, i2)
        gbufs = (g0, g1, g2)
        sems = (s0, s1, s2)
        outs = (o0, o1, o2)

        for f in range(3):
            pltpu.sync_copy(xt_hbm.at[f, pl.ds(base, BPW)], xfs[f])

        for f in range(3):
            def conv_body(g, f=f):
                v = xfs[f][pl.ds(g * L, L)].astype(jnp.int32)
                b2 = lax.shift_left(lax.shift_right_logical(v, 7), 10) + (v & 127)
                for d in range(D8):
                    idxs[f].at[d, g // 8][pl.ds((g % 8) * L, L)] = b2 + d * CHUNK
            pl.loop(0, NGRP)(conv_body)

        handles = []
        for f in range(3):
            for d in range(D8):
                for j in range(NCHUNK):
                    handles.append(pltpu.async_copy(
                        tabs[f].at[idxs[f].at[d, j]],
                        gbufs[f].at[d, pl.ds(j * CHUNK, CHUNK)],
                        sems[f]))
        for h in handles:
            h.wait()
        for f in range(3):
            pltpu.sync_copy(gbufs[f], outs[f].at[:, pl.ds(base, BPW)])

    return k(xt, t0f, t1f, t2f)


def _tc_small_body(xt, s0, s1, s2, s3, s4, s5, w1t, b1c, zout):
    small = (s0, s1, s2, s3, s4, s5)
    z = lax.dot_general(w1t[45:46, :], xt[9:10, :],
                        (((0,), (0,)), ((), ())),
                        preferred_element_type=jnp.float32)
    off = 24
    for f in range(6):
        v = SMALL_VOCABS[f]
        d = DIMS[3 + f]
        proj = lax.dot_general(small[f][...], w1t[off:off + d, :],
                               (((0,), (0,)), ((), ())),
                               preferred_element_type=jnp.float32)
        ids = xt[3 + f:4 + f, :].astype(jnp.int32)
        onehot = (lax.broadcasted_iota(jnp.int32, (v, 1), 0) == ids
                  ).astype(jnp.float32)
        z = z + lax.dot_general(proj, onehot, (((0,), (0,)), ((), ())),
                                preferred_element_type=jnp.float32)
        off += d
    zout[...] = z + b1c[...]


def _tc_mlp_body(zs, g0, g1, g2, w1t, w2, b2, out):
    z = zs[...] + lax.dot_general(w1t[0:8, :], g0[...],
                                  (((0,), (0,)), ((), ())),
                                  preferred_element_type=jnp.float32)
    z = z + lax.dot_general(w1t[8:16, :], g1[...], (((0,), (0,)), ((), ())),
                            preferred_element_type=jnp.float32)
    z = z + lax.dot_general(w1t[16:24, :], g2[...], (((0,), (0,)), ((), ())),
                            preferred_element_type=jnp.float32)
    h1 = jnp.maximum(z, 0.0)
    o = jnp.dot(w2[...], h1, preferred_element_type=jnp.float32) + b2[...]
    out[...] = jax.nn.sigmoid(o)


def kernel(x, W_emb0, W_emb1, W_emb2, W_emb3, W_emb4, W_emb5, W_emb6,
           W_emb7, W_emb8, W1, b1, W2, b2):
    xt = x.T
    r0, r1, r2 = _sc_repack(W_emb0.T, W_emb1.T, W_emb2.T)
    g0, g1, g2 = _sc_gather(xt, r0.reshape(-1), r1.reshape(-1), r2.reshape(-1))

    w1t = W1.T
    BB = 2048
    col_blk = lambda h: pl.BlockSpec((h, BB), lambda i: (0, i))
    full = lambda s: pl.BlockSpec(s, lambda i: (0, 0))
    zs = pl.pallas_call(
        _tc_small_body,
        grid=(B // BB,),
        in_specs=[col_blk(10),
                  full((2, 3)), full((4, 8)), full((3, 4)), full((4, 16)),
                  full((4, 8)), full((4, 16)),
                  full((46, HIDDEN)), full((HIDDEN, 1))],
        out_specs=col_blk(HIDDEN),
        out_shape=jax.ShapeDtypeStruct((HIDDEN, B), jnp.float32),
    )(xt, W_emb3.T, W_emb4.T, W_emb5.T, W_emb6.T, W_emb7.T, W_emb8.T,
      w1t, b1.reshape(HIDDEN, 1))
    out = pl.pallas_call(
        _tc_mlp_body,
        grid=(B // BB,),
        in_specs=[col_blk(HIDDEN), col_blk(8), col_blk(8), col_blk(8),
                  full((46, HIDDEN)), full((1, HIDDEN)), full((1, 1))],
        out_specs=col_blk(1),
        out_shape=jax.ShapeDtypeStruct((1, B), jnp.float32),
    )(zs, g0, g1, g2, w1t, W2, b2.reshape(1, 1))
    return jnp.reshape(out, (B,))
